# baseline (speedup 1.0000x reference)
import functools
import jax
import jax.numpy as jnp
from jax import lax
from jax.experimental import pallas as pl
from jax.experimental.pallas import tpu as pltpu
from jax.experimental.pallas import tpu_sc as plsc

N = 10000
E = 320000
D_IN = 128
H = 64
C = 3

NC, NS = 2, 16
NW = NC * NS
NPAD = 10240
EB = 128
SB = 1
SBW = SB * EB
NSB = 80
TOTSB = NW * NSB
EPAD = TOTSB * SBW
RPT = NPAD // NS

_mesh = plsc.VectorSubcoreMesh(core_axis_name="c", subcore_axis_name="s")
_sc_params = pltpu.CompilerParams(use_tc_tiling_on_sc=False)


def _deg_body(dst_hbm, ew_hbm, out_hbm, didx, ewb, stripe_v, acc, sem, lsem):
    c = lax.axis_index("c")
    s = lax.axis_index("s")
    start = (c * NS + s) * NSB

    def _zero(i, _):
        stripe_v[pl.ds(i * 16, 16)] = jnp.zeros((16,), jnp.float32)
        return 0

    lax.fori_loop(0, RPT // 16, _zero, 0)
    pltpu.sync_copy(stripe_v, acc.at[pl.ds(s * RPT, RPT)])
    pltpu.async_copy(dst_hbm.at[pl.ds(start, NSB)], didx, lsem)
    pltpu.async_copy(ew_hbm.at[pl.ds(start, NSB)], ewb, lsem)
    pltpu.make_async_copy(dst_hbm.at[pl.ds(start, NSB)], didx, lsem).wait()
    pltpu.make_async_copy(ew_hbm.at[pl.ds(start, NSB)], ewb, lsem).wait()
    plsc.subcore_barrier()

    K = 8

    def _grp(g, _):
        b0 = g * K
        cps = [
            pltpu.async_copy(ewb.at[b0 + j], acc.at[didx.at[b0 + j]],
                             sem, add=True)
            for j in range(K)
        ]
        for cp in cps:
            cp.wait()
        return 0

    lax.fori_loop(0, NSB // K, _grp, 0)
    plsc.subcore_barrier()

    pltpu.sync_copy(acc.at[pl.ds(s * RPT, RPT)], stripe_v)
    pltpu.sync_copy(stripe_v, out_hbm.at[c, pl.ds(s * RPT, RPT)])


_deg_call = pl.kernel(
    _deg_body,
    out_type=jax.ShapeDtypeStruct((NC, NPAD), jnp.float32),
    mesh=_mesh,
    scratch_types=[
        pltpu.VMEM((NSB, SBW), jnp.int32),
        pltpu.VMEM((NSB, SBW), jnp.float32),
        pltpu.VMEM((RPT,), jnp.float32),
        pltpu.VMEM_SHARED((NPAD,), jnp.float32),
        pltpu.SemaphoreType.DMA,
        pltpu.SemaphoreType.DMA,
    ],
    compiler_params=_sc_params,
)


def _agg_body(ys_hbm, src_hbm, dst_hbm, ew_hbm, out_hbm,
              sidx, didx, ewb, rows0, rows1, rows2, rows3, acc,
              g0, g1, g2, g3, s0, s1, s2, s3, lsem):
    c = lax.axis_index("c")
    s = lax.axis_index("s")
    start = (c * NS + s) * NSB

    def _zero(i, _):
        for j in range(4):
            rows0[i, pl.ds(j * 16, 16)] = jnp.zeros((16,), jnp.float32)
        return 0

    lax.fori_loop(0, SBW, _zero, 0)
    pltpu.async_copy(src_hbm.at[pl.ds(start, NSB)],
                     sidx.at[pl.ds(0, NSB)], lsem)
    pltpu.async_copy(dst_hbm.at[pl.ds(start, NSB)], didx, lsem)
    pltpu.async_copy(ew_hbm.at[pl.ds(start, NSB)], ewb, lsem)
    for k in range(RPT // SBW):
        pltpu.sync_copy(rows0, acc.at[pl.ds(s * RPT + k * SBW, SBW)])
    if RPT % SBW:
        pltpu.sync_copy(rows0.at[pl.ds(0, RPT % SBW)],
                        acc.at[pl.ds(s * RPT + (RPT // SBW) * SBW,
                                     RPT % SBW)])
    pltpu.make_async_copy(src_hbm.at[pl.ds(start, NSB)],
                          sidx.at[pl.ds(0, NSB)], lsem).wait()
    pltpu.make_async_copy(dst_hbm.at[pl.ds(start, NSB)], didx, lsem).wait()
    pltpu.make_async_copy(ew_hbm.at[pl.ds(start, NSB)], ewb, lsem).wait()
    for d in range(2):
        for j in range(SBW // 16):
            sidx[NSB + d, pl.ds(j * 16, 16)] = jnp.zeros((16,), jnp.int32)
    plsc.subcore_barrier()

    bufs = (rows0, rows1, rows2, rows3)
    gsems = (g0, g1, g2, g3)
    ssems = (s0, s1, s2, s3)

    def _gather(sb, k):
        pltpu.async_copy(ys_hbm.at[sidx.at[sb]], bufs[k], gsems[k])

    def _gwait(sb, k):
        pltpu.make_async_copy(ys_hbm.at[sidx.at[sb]], bufs[k],
                              gsems[k]).wait()

    def _scatter(sb, k):
        pltpu.async_copy(bufs[k], acc.at[didx.at[sb]], ssems[k], add=True)

    def _swait(sb, k):
        pltpu.make_async_copy(bufs[k], acc.at[didx.at[sb]], ssems[k]).wait()

    def _scale(sb, k):
        buf = bufs[k]

        def _grp(g, _):
            wv = ewb[sb, pl.ds(g * 16, 16)]
            for l in range(16):
                sw = wv[l]
                r = g * 16 + l
                for j in range(4):
                    buf[r, pl.ds(j * 16, 16)] = buf[r, pl.ds(j * 16, 16)] * sw
            return 0

        lax.fori_loop(0, SBW // 16, _grp, 0)

    def _step(b, k, first):
        _gwait(b, k)
        _scale(b, k)
        _scatter(b, k)
        kn = (k + 2) % 4
        if not first:
            _swait(b - 2, kn)
        _gather(b + 2, kn)

    _gather(0, 0)
    _gather(1, 1)
    for b in range(4):
        _step(b, b, first=(b < 2))

    def _quad(gq, _):
        b0 = gq * 4
        for k in range(4):
            _step(b0 + k, k, first=False)
        return 0

    lax.fori_loop(1, NSB // 4, _quad, 0)
    _gwait(NSB, 0)
    _gwait(NSB + 1, 1)
    _swait(NSB - 2, 2)
    _swait(NSB - 1, 3)
    plsc.subcore_barrier()

    for k in range(RPT // SBW):
        r0 = s * RPT + k * SBW
        pltpu.sync_copy(acc.at[pl.ds(r0, SBW)], rows0)
        pltpu.sync_copy(rows0, out_hbm.at[c, pl.ds(r0, SBW)])
    if RPT % SBW:
        rem = RPT % SBW
        r0 = s * RPT + (RPT // SBW) * SBW
        pltpu.sync_copy(acc.at[pl.ds(r0, rem)], rows0.at[pl.ds(0, rem)])
        pltpu.sync_copy(rows0.at[pl.ds(0, rem)],
                        out_hbm.at[c, pl.ds(r0, rem)])


_agg_call = pl.kernel(
    _agg_body,
    out_type=jax.ShapeDtypeStruct((NC, NPAD, H), jnp.float32),
    mesh=_mesh,
    scratch_types=[
        pltpu.VMEM((NSB + 2, SBW), jnp.int32),
        pltpu.VMEM((NSB, SBW), jnp.int32),
        pltpu.VMEM((NSB, SBW), jnp.float32),
        pltpu.VMEM((SBW, H), jnp.float32),
        pltpu.VMEM((SBW, H), jnp.float32),
        pltpu.VMEM((SBW, H), jnp.float32),
        pltpu.VMEM((SBW, H), jnp.float32),
        pltpu.VMEM_SHARED((NPAD, H), jnp.float32),
        pltpu.SemaphoreType.DMA,
        pltpu.SemaphoreType.DMA,
        pltpu.SemaphoreType.DMA,
        pltpu.SemaphoreType.DMA,
        pltpu.SemaphoreType.DMA,
        pltpu.SemaphoreType.DMA,
        pltpu.SemaphoreType.DMA,
        pltpu.SemaphoreType.DMA,
        pltpu.SemaphoreType.DMA,
    ],
    compiler_params=_sc_params,
)


_RB = 2048
_GRID = NPAD // _RB


def _mm1_body(x_ref, w_ref, deg_ref, o_ref, dinv_ref):
    dinv = lax.rsqrt(deg_ref[0] + deg_ref[1] + 1.0)
    dinv_ref[...] = dinv
    o_ref[...] = jnp.dot(x_ref[...], w_ref[...],
                         preferred_element_type=jnp.float32) * dinv


def _mm2_body(p_ref, ys_ref, b_ref, w_ref, dinv_ref, o_ref):
    dinv = dinv_ref[...]
    h = jnp.maximum(
        dinv * (p_ref[0] + p_ref[1] + ys_ref[...]) + b_ref[...], 0.0)
    o_ref[...] = jnp.dot(h, w_ref[...],
                         preferred_element_type=jnp.float32) * dinv


def _mm3_body(p_ref, ys_ref, b_ref, wc1_ref, bc1_ref, wc2_ref, bc2_ref,
              dinv_ref, o_ref):
    dinv = dinv_ref[...]
    h2 = jnp.maximum(
        dinv * (p_ref[0] + p_ref[1] + ys_ref[...]) + b_ref[...], 0.0)
    t = jnp.maximum(
        jnp.dot(h2, wc1_ref[...], preferred_element_type=jnp.float32)
        + bc1_ref[...], 0.0)
    o_ref[...] = jnp.dot(t, wc2_ref[...],
                         preferred_element_type=jnp.float32) + bc2_ref[...]


_mm1 = pl.pallas_call(
    _mm1_body,
    grid=(_GRID,),
    in_specs=[
        pl.BlockSpec((_RB, D_IN), lambda i: (i, 0)),
        pl.BlockSpec((D_IN, H), lambda i: (0, 0)),
        pl.BlockSpec((NC, _RB, 1), lambda i: (0, i, 0)),
    ],
    out_specs=[
        pl.BlockSpec((_RB, H), lambda i: (i, 0)),
        pl.BlockSpec((_RB, 1), lambda i: (i, 0)),
    ],
    out_shape=[
        jax.ShapeDtypeStruct((N, H), jnp.float32),
        jax.ShapeDtypeStruct((NPAD, 1), jnp.float32),
    ],
)

_mm2 = pl.pallas_call(
    _mm2_body,
    grid=(_GRID,),
    in_specs=[
        pl.BlockSpec((NC, _RB, H), lambda i: (0, i, 0)),
        pl.BlockSpec((_RB, H), lambda i: (i, 0)),
        pl.BlockSpec((1, H), lambda i: (0, 0)),
        pl.BlockSpec((H, H), lambda i: (0, 0)),
        pl.BlockSpec((_RB, 1), lambda i: (i, 0)),
    ],
    out_specs=pl.BlockSpec((_RB, H), lambda i: (i, 0)),
    out_shape=jax.ShapeDtypeStruct((N, H), jnp.float32),
)

_mm3 = pl.pallas_call(
    _mm3_body,
    grid=(_GRID,),
    in_specs=[
        pl.BlockSpec((NC, _RB, H), lambda i: (0, i, 0)),
        pl.BlockSpec((_RB, H), lambda i: (i, 0)),
        pl.BlockSpec((1, H), lambda i: (0, 0)),
        pl.BlockSpec((H, H // 2), lambda i: (0, 0)),
        pl.BlockSpec((1, H // 2), lambda i: (0, 0)),
        pl.BlockSpec((H // 2, C), lambda i: (0, 0)),
        pl.BlockSpec((1, C), lambda i: (0, 0)),
        pl.BlockSpec((_RB, 1), lambda i: (i, 0)),
    ],
    out_specs=pl.BlockSpec((_RB, C), lambda i: (i, 0)),
    out_shape=jax.ShapeDtypeStruct((N, C), jnp.float32),
)


@jax.jit
def kernel(x, edge_index, edge_attr, W1, b1, W2, b2, Wc1, bc1, Wc2, bc2):
    src = edge_index[0]
    dst = edge_index[1]
    ew = jnp.squeeze(edge_attr, axis=-1)

    pad = EPAD - E
    srcp = jnp.concatenate(
        [src, jnp.zeros((pad,), src.dtype)]).reshape(TOTSB, SBW)
    dstp = jnp.concatenate(
        [dst, jnp.zeros((pad,), dst.dtype)]).reshape(TOTSB, SBW)
    ewp = jnp.concatenate(
        [ew, jnp.zeros((pad,), ew.dtype)]).reshape(TOTSB, SBW)

    deg_parts = _deg_call(dstp, ewp)
    ys1, dinv = _mm1(x, W1, deg_parts.reshape(NC, NPAD, 1))
    p1 = _agg_call(ys1, srcp, dstp, ewp)
    ys2 = _mm2(p1, ys1, b1.reshape(1, H), W2, dinv)
    p2 = _agg_call(ys2, srcp, dstp, ewp)
    out = _mm3(p2, ys2, b2.reshape(1, H), Wc1,
               bc1.reshape(1, H // 2), Wc2, bc2.reshape(1, C), dinv)
    return out


# speedup vs baseline: 2.4969x; 2.4969x over previous
import functools
import jax
import jax.numpy as jnp
from jax import lax
from jax.experimental import pallas as pl
from jax.experimental.pallas import tpu as pltpu
from jax.experimental.pallas import tpu_sc as plsc

N = 10000
E = 320000
D_IN = 128
H = 64
C = 3

NC, NS = 2, 16
NW = NC * NS
NPAD = 10240
EB = 128
SB = 1
SBW = SB * EB
NSB = 80
TOTSB = NW * NSB
EPAD = TOTSB * SBW
RPT = NPAD // NS

_mesh = plsc.VectorSubcoreMesh(core_axis_name="c", subcore_axis_name="s")
_sc_params = pltpu.CompilerParams(use_tc_tiling_on_sc=False)


def _deg_body(dst_hbm, ew_hbm, out_hbm, didx, ewb, stripe_v, acc, sem, lsem):
    c = lax.axis_index("c")
    s = lax.axis_index("s")
    start = (c * NS + s) * NSB

    def _zero(i, _):
        stripe_v[pl.ds(i * 16, 16)] = jnp.zeros((16,), jnp.float32)
        return 0

    lax.fori_loop(0, RPT // 16, _zero, 0)
    pltpu.sync_copy(stripe_v, acc.at[pl.ds(s * RPT, RPT)])
    pltpu.async_copy(dst_hbm.at[pl.ds(start, NSB)], didx, lsem)
    pltpu.async_copy(ew_hbm.at[pl.ds(start, NSB)], ewb, lsem)
    pltpu.make_async_copy(dst_hbm.at[pl.ds(start, NSB)], didx, lsem).wait()
    pltpu.make_async_copy(ew_hbm.at[pl.ds(start, NSB)], ewb, lsem).wait()
    plsc.subcore_barrier()

    K = 8

    def _grp(g, _):
        b0 = g * K
        cps = [
            pltpu.async_copy(ewb.at[b0 + j], acc.at[didx.at[b0 + j]],
                             sem, add=True)
            for j in range(K)
        ]
        for cp in cps:
            cp.wait()
        return 0

    lax.fori_loop(0, NSB // K, _grp, 0)
    plsc.subcore_barrier()

    pltpu.sync_copy(acc.at[pl.ds(s * RPT, RPT)], stripe_v)
    pltpu.sync_copy(stripe_v, out_hbm.at[c, pl.ds(s * RPT, RPT)])


_deg_call = pl.kernel(
    _deg_body,
    out_type=jax.ShapeDtypeStruct((NC, NPAD), jnp.float32),
    mesh=_mesh,
    scratch_types=[
        pltpu.VMEM((NSB, SBW), jnp.int32),
        pltpu.VMEM((NSB, SBW), jnp.float32),
        pltpu.VMEM((RPT,), jnp.float32),
        pltpu.VMEM_SHARED((NPAD,), jnp.float32),
        pltpu.SemaphoreType.DMA,
        pltpu.SemaphoreType.DMA,
    ],
    compiler_params=_sc_params,
)


NSTG = N // NS


def _agg_body(ys_hbm, src_hbm, dst_hbm, ew_hbm, out_hbm,
              sidx, didx, ewb, rows0, rows1, ys_sh, acc, g0, g1, lsem):
    c = lax.axis_index("c")
    s = lax.axis_index("s")
    start = (c * NS + s) * NSB

    def _zero(i, _):
        for j in range(4):
            rows0[i, pl.ds(j * 16, 16)] = jnp.zeros((16,), jnp.float32)
        return 0

    lax.fori_loop(0, SBW, _zero, 0)
    pltpu.async_copy(ys_hbm.at[pl.ds(s * NSTG, NSTG)],
                     ys_sh.at[pl.ds(s * NSTG, NSTG)], lsem)
    pltpu.async_copy(src_hbm.at[pl.ds(start, NSB)],
                     sidx.at[pl.ds(0, NSB)], lsem)
    pltpu.async_copy(dst_hbm.at[pl.ds(start, NSB)], didx, lsem)
    pltpu.async_copy(ew_hbm.at[pl.ds(start, NSB)], ewb, lsem)
    for k in range(RPT // SBW):
        pltpu.sync_copy(rows0, acc.at[pl.ds(s * RPT + k * SBW, SBW)])
    if RPT % SBW:
        pltpu.sync_copy(rows0.at[pl.ds(0, RPT % SBW)],
                        acc.at[pl.ds(s * RPT + (RPT // SBW) * SBW,
                                     RPT % SBW)])
    pltpu.make_async_copy(ys_hbm.at[pl.ds(s * NSTG, NSTG)],
                          ys_sh.at[pl.ds(s * NSTG, NSTG)], lsem).wait()
    pltpu.make_async_copy(src_hbm.at[pl.ds(start, NSB)],
                          sidx.at[pl.ds(0, NSB)], lsem).wait()
    pltpu.make_async_copy(dst_hbm.at[pl.ds(start, NSB)], didx, lsem).wait()
    pltpu.make_async_copy(ew_hbm.at[pl.ds(start, NSB)], ewb, lsem).wait()
    for j in range(SBW // 16):
        sidx[NSB, pl.ds(j * 16, 16)] = jnp.zeros((16,), jnp.int32)
    plsc.subcore_barrier()

    bufs = (rows0, rows1)
    gsems = (g0, g1)

    def _gather(sb, k):
        pltpu.async_copy(ys_sh.at[sidx.at[sb]], bufs[k], gsems[k])

    def _gwait(sb, k):
        pltpu.make_async_copy(ys_sh.at[sidx.at[sb]], bufs[k],
                              gsems[k]).wait()

    def _scale(sb, k):
        buf = bufs[k]

        def _grp(g, _):
            wv = ewb[sb, pl.ds(g * 16, 16)]
            for l in range(16):
                sw = wv[l]
                r = g * 16 + l
                for j in range(4):
                    buf[r, pl.ds(j * 16, 16)] = buf[r, pl.ds(j * 16, 16)] * sw
            return 0

        lax.fori_loop(0, SBW // 16, _grp, 0)

    _gather(0, 0)

    def _pair(i, _):
        sb = 2 * i
        for k in range(2):
            sbk = sb + k
            _gather(sbk + 1, (k + 1) % 2)
            _gwait(sbk, k)
            _scale(sbk, k)
            pltpu.sync_copy(bufs[k], acc.at[didx.at[sbk]], add=True)
        return 0

    lax.fori_loop(0, NSB // 2, _pair, 0)
    _gwait(NSB, 0)
    plsc.subcore_barrier()

    for k in range(RPT // SBW):
        r0 = s * RPT + k * SBW
        pltpu.sync_copy(acc.at[pl.ds(r0, SBW)], rows0)
        pltpu.sync_copy(rows0, out_hbm.at[c, pl.ds(r0, SBW)])
    if RPT % SBW:
        rem = RPT % SBW
        r0 = s * RPT + (RPT // SBW) * SBW
        pltpu.sync_copy(acc.at[pl.ds(r0, rem)], rows0.at[pl.ds(0, rem)])
        pltpu.sync_copy(rows0.at[pl.ds(0, rem)],
                        out_hbm.at[c, pl.ds(r0, rem)])


_agg_call = pl.kernel(
    _agg_body,
    out_type=jax.ShapeDtypeStruct((NC, NPAD, H), jnp.float32),
    mesh=_mesh,
    scratch_types=[
        pltpu.VMEM((NSB + 1, SBW), jnp.int32),
        pltpu.VMEM((NSB, SBW), jnp.int32),
        pltpu.VMEM((NSB, SBW), jnp.float32),
        pltpu.VMEM((SBW, H), jnp.float32),
        pltpu.VMEM((SBW, H), jnp.float32),
        pltpu.VMEM_SHARED((NPAD, H), jnp.float32),
        pltpu.VMEM_SHARED((NPAD, H), jnp.float32),
        pltpu.SemaphoreType.DMA,
        pltpu.SemaphoreType.DMA,
        pltpu.SemaphoreType.DMA,
    ],
    compiler_params=_sc_params,
)


_RB = 2048
_GRID = NPAD // _RB


def _mm1_body(x_ref, w_ref, deg_ref, o_ref, dinv_ref):
    dinv = lax.rsqrt(deg_ref[0] + deg_ref[1] + 1.0)
    dinv_ref[...] = dinv
    o_ref[...] = jnp.dot(x_ref[...], w_ref[...],
                         preferred_element_type=jnp.float32) * dinv


def _mm2_body(p_ref, ys_ref, b_ref, w_ref, dinv_ref, o_ref):
    dinv = dinv_ref[...]
    h = jnp.maximum(
        dinv * (p_ref[0] + p_ref[1] + ys_ref[...]) + b_ref[...], 0.0)
    o_ref[...] = jnp.dot(h, w_ref[...],
                         preferred_element_type=jnp.float32) * dinv


def _mm3_body(p_ref, ys_ref, b_ref, wc1_ref, bc1_ref, wc2_ref, bc2_ref,
              dinv_ref, o_ref):
    dinv = dinv_ref[...]
    h2 = jnp.maximum(
        dinv * (p_ref[0] + p_ref[1] + ys_ref[...]) + b_ref[...], 0.0)
    t = jnp.maximum(
        jnp.dot(h2, wc1_ref[...], preferred_element_type=jnp.float32)
        + bc1_ref[...], 0.0)
    o_ref[...] = jnp.dot(t, wc2_ref[...],
                         preferred_element_type=jnp.float32) + bc2_ref[...]


_mm1 = pl.pallas_call(
    _mm1_body,
    grid=(_GRID,),
    in_specs=[
        pl.BlockSpec((_RB, D_IN), lambda i: (i, 0)),
        pl.BlockSpec((D_IN, H), lambda i: (0, 0)),
        pl.BlockSpec((NC, _RB, 1), lambda i: (0, i, 0)),
    ],
    out_specs=[
        pl.BlockSpec((_RB, H), lambda i: (i, 0)),
        pl.BlockSpec((_RB, 1), lambda i: (i, 0)),
    ],
    out_shape=[
        jax.ShapeDtypeStruct((N, H), jnp.float32),
        jax.ShapeDtypeStruct((NPAD, 1), jnp.float32),
    ],
)

_mm2 = pl.pallas_call(
    _mm2_body,
    grid=(_GRID,),
    in_specs=[
        pl.BlockSpec((NC, _RB, H), lambda i: (0, i, 0)),
        pl.BlockSpec((_RB, H), lambda i: (i, 0)),
        pl.BlockSpec((1, H), lambda i: (0, 0)),
        pl.BlockSpec((H, H), lambda i: (0, 0)),
        pl.BlockSpec((_RB, 1), lambda i: (i, 0)),
    ],
    out_specs=pl.BlockSpec((_RB, H), lambda i: (i, 0)),
    out_shape=jax.ShapeDtypeStruct((N, H), jnp.float32),
)

_mm3 = pl.pallas_call(
    _mm3_body,
    grid=(_GRID,),
    in_specs=[
        pl.BlockSpec((NC, _RB, H), lambda i: (0, i, 0)),
        pl.BlockSpec((_RB, H), lambda i: (i, 0)),
        pl.BlockSpec((1, H), lambda i: (0, 0)),
        pl.BlockSpec((H, H // 2), lambda i: (0, 0)),
        pl.BlockSpec((1, H // 2), lambda i: (0, 0)),
        pl.BlockSpec((H // 2, C), lambda i: (0, 0)),
        pl.BlockSpec((1, C), lambda i: (0, 0)),
        pl.BlockSpec((_RB, 1), lambda i: (i, 0)),
    ],
    out_specs=pl.BlockSpec((_RB, C), lambda i: (i, 0)),
    out_shape=jax.ShapeDtypeStruct((N, C), jnp.float32),
)


@jax.jit
def kernel(x, edge_index, edge_attr, W1, b1, W2, b2, Wc1, bc1, Wc2, bc2):
    src = edge_index[0]
    dst = edge_index[1]
    ew = jnp.squeeze(edge_attr, axis=-1)

    pad = EPAD - E
    srcp = jnp.concatenate(
        [src, jnp.zeros((pad,), src.dtype)]).reshape(TOTSB, SBW)
    dstp = jnp.concatenate(
        [dst, jnp.zeros((pad,), dst.dtype)]).reshape(TOTSB, SBW)
    ewp = jnp.concatenate(
        [ew, jnp.zeros((pad,), ew.dtype)]).reshape(TOTSB, SBW)

    deg_parts = _deg_call(dstp, ewp)
    ys1, dinv = _mm1(x, W1, deg_parts.reshape(NC, NPAD, 1))
    p1 = _agg_call(ys1, srcp, dstp, ewp)
    ys2 = _mm2(p1, ys1, b1.reshape(1, H), W2, dinv)
    p2 = _agg_call(ys2, srcp, dstp, ewp)
    out = _mm3(p2, ys2, b2.reshape(1, H), Wc1,
               bc1.reshape(1, H // 2), Wc2, bc2.reshape(1, C), dinv)
    return out
